# Initial kernel scaffold; baseline (speedup 1.0000x reference)
import jax
import jax.numpy as jnp
from jax.experimental import pallas as pl


def kernel(value_ids, edge_type_ids, value_table, W_proj, b_proj, ln_gamma, ln_beta, edge_type_table):
    raise NotImplementedError("write your pallas kernel here")


# speedup vs baseline: 4.2999x; 4.2999x over previous
import functools

import jax
import jax.numpy as jnp
from jax import lax
from jax.experimental import pallas as pl
from jax.experimental.pallas import tpu as pltpu
from jax.experimental.pallas import tpu_sc as plsc

_NUM_WORKERS = 32
_VCHUNK = 64


def _sc_gather(value_table, vids2, B, H):
    rows_per_w = B // _NUM_WORKERS
    nvc = rows_per_w // _VCHUNK
    mesh = plsc.VectorSubcoreMesh(core_axis_name="c", subcore_axis_name="s")

    @functools.partial(
        pl.kernel,
        mesh=mesh,
        out_type=jax.ShapeDtypeStruct((B, H), jnp.float32),
        scratch_types=[
            pltpu.VMEM((nvc, _VCHUNK), jnp.int32),
            pltpu.VMEM((_VCHUNK, H), jnp.float32),
            pltpu.VMEM((_VCHUNK, H), jnp.float32),
            pltpu.SemaphoreType.DMA,
            pltpu.SemaphoreType.DMA,
        ],
    )
    def k(table_hbm, vids_hbm, emb_hbm, vidx, rows0, rows1, sem0, sem1):
        wid = lax.axis_index("s") * 2 + lax.axis_index("c")
        pltpu.sync_copy(vids_hbm.at[pl.ds(wid * nvc, nvc)], vidx)
        base = wid * rows_per_w
        bufs = (rows0, rows1)
        sems = (sem0, sem1)
        copies = [
            pltpu.make_async_copy(table_hbm.at[vidx.at[c]], bufs[c % 2], sems[c % 2])
            for c in range(nvc)
        ]
        copies[0].start()
        for c in range(nvc):
            copies[c].wait()
            if c + 1 < nvc:
                copies[c + 1].start()
            pltpu.sync_copy(bufs[c % 2], emb_hbm.at[pl.ds(base + c * _VCHUNK, _VCHUNK)])

    return k(value_table, vids2)


def _tc_dense(emb, W, b, g, be, eids2, etab):
    Bdim, H = emb.shape
    n_types, EA = etab.shape
    blk = 512

    def body(emb_ref, w_ref, b_ref, g_ref, be_ref, eid_ref, etab_ref,
             out_ref, eattr_ref):
        h = jnp.dot(emb_ref[...], w_ref[...], preferred_element_type=jnp.float32)
        h = h + b_ref[...]
        mean = jnp.mean(h, axis=-1, keepdims=True)
        var = jnp.mean((h - mean) ** 2, axis=-1, keepdims=True)
        out_ref[...] = (h - mean) * lax.rsqrt(var + 1e-5) * g_ref[...] + be_ref[...]
        ids = eid_ref[0, 0, :][:, None]
        onehot = (ids == lax.broadcasted_iota(jnp.int32, (blk, n_types), 1))
        eattr_ref[...] = jnp.dot(onehot.astype(jnp.float32), etab_ref[...],
                                 preferred_element_type=jnp.float32)

    return pl.pallas_call(
        body,
        grid=(Bdim // blk,),
        in_specs=[
            pl.BlockSpec((blk, H), lambda i: (i, 0)),
            pl.BlockSpec((H, H), lambda i: (0, 0)),
            pl.BlockSpec((1, H), lambda i: (0, 0)),
            pl.BlockSpec((1, H), lambda i: (0, 0)),
            pl.BlockSpec((1, H), lambda i: (0, 0)),
            pl.BlockSpec((1, 1, blk), lambda i: (i, 0, 0)),
            pl.BlockSpec((n_types, EA), lambda i: (0, 0)),
        ],
        out_specs=[
            pl.BlockSpec((blk, H), lambda i: (i, 0)),
            pl.BlockSpec((blk, EA), lambda i: (i, 0)),
        ],
        out_shape=[
            jax.ShapeDtypeStruct((Bdim, H), jnp.float32),
            jax.ShapeDtypeStruct((eids2.size, EA), jnp.float32),
        ],
    )(emb, W, b.reshape(1, H), g.reshape(1, H), be.reshape(1, H), eids2, etab)


def kernel(value_ids, edge_type_ids, value_table, W_proj, b_proj, ln_gamma,
           ln_beta, edge_type_table):
    B = value_ids.shape[0]
    E = edge_type_ids.shape[0]
    H = value_table.shape[1]
    vids2 = value_ids.astype(jnp.int32).reshape(B // _VCHUNK, _VCHUNK)
    eids2 = edge_type_ids.astype(jnp.int32).reshape(E // 512, 1, 512)
    emb = _sc_gather(value_table, vids2, B, H)
    node_feat, eattr = _tc_dense(emb, W_proj, b_proj, ln_gamma, ln_beta,
                                 eids2, edge_type_table)
    return node_feat, eattr


# speedup vs baseline: 4.4820x; 1.0424x over previous
import functools

import jax
import jax.numpy as jnp
from jax import lax
from jax.experimental import pallas as pl
from jax.experimental.pallas import tpu as pltpu
from jax.experimental.pallas import tpu_sc as plsc

_NUM_WORKERS = 32
_VCHUNK = 64
_NCHUNKS = 4
_BLK = 512


def _sc_gather(value_table, vids2, Bc, H):
    rows_per_w = Bc // _NUM_WORKERS
    nvc = rows_per_w // _VCHUNK
    mesh = plsc.VectorSubcoreMesh(core_axis_name="c", subcore_axis_name="s")

    @functools.partial(
        pl.kernel,
        mesh=mesh,
        out_type=jax.ShapeDtypeStruct((Bc, H), jnp.float32),
        scratch_types=[
            pltpu.VMEM((nvc, _VCHUNK), jnp.int32),
            pltpu.VMEM((_VCHUNK, H), jnp.float32),
            pltpu.VMEM((_VCHUNK, H), jnp.float32),
            pltpu.SemaphoreType.DMA,
            pltpu.SemaphoreType.DMA,
        ],
    )
    def k(table_hbm, vids_hbm, emb_hbm, vidx, rows0, rows1, sem0, sem1):
        wid = lax.axis_index("s") * 2 + lax.axis_index("c")
        pltpu.sync_copy(vids_hbm.at[pl.ds(wid * nvc, nvc)], vidx)
        base = wid * rows_per_w
        bufs = (rows0, rows1)
        sems = (sem0, sem1)
        copies = [
            pltpu.make_async_copy(table_hbm.at[vidx.at[c]], bufs[c % 2], sems[c % 2])
            for c in range(nvc)
        ]
        copies[0].start()
        for c in range(nvc):
            copies[c].wait()
            if c + 1 < nvc:
                copies[c + 1].start()
            pltpu.sync_copy(bufs[c % 2], emb_hbm.at[pl.ds(base + c * _VCHUNK, _VCHUNK)])

    return k(value_table, vids2)


def _tc_dense(emb_c, W, b, g, be, eids_c, etab, node_acc, eattr_acc, chunk, Btot):
    Bc, H = emb_c.shape
    n_types, EA = etab.shape
    nsteps = Bc // _BLK
    aliased = node_acc is not None

    def body(*refs):
        if aliased:
            refs = refs[2:]
        (emb_ref, w_ref, b_ref, g_ref, be_ref, eid_ref, etab_ref,
         out_ref, eattr_ref) = refs
        h = jnp.dot(emb_ref[...], w_ref[...], preferred_element_type=jnp.float32)
        h = h + b_ref[...]
        mean = jnp.mean(h, axis=-1, keepdims=True)
        var = jnp.mean((h - mean) ** 2, axis=-1, keepdims=True)
        out_ref[...] = (h - mean) * lax.rsqrt(var + 1e-5) * g_ref[...] + be_ref[...]
        ids = eid_ref[0, 0, :][:, None]
        onehot = (ids == lax.broadcasted_iota(jnp.int32, (_BLK, n_types), 1))
        eattr_ref[...] = jnp.dot(onehot.astype(jnp.float32), etab_ref[...],
                                 preferred_element_type=jnp.float32)

    base = chunk * nsteps
    in_specs = [
        pl.BlockSpec((_BLK, H), lambda i: (i, 0)),
        pl.BlockSpec((H, H), lambda i: (0, 0)),
        pl.BlockSpec((1, H), lambda i: (0, 0)),
        pl.BlockSpec((1, H), lambda i: (0, 0)),
        pl.BlockSpec((1, H), lambda i: (0, 0)),
        pl.BlockSpec((1, 1, _BLK), lambda i: (i, 0, 0)),
        pl.BlockSpec((n_types, EA), lambda i: (0, 0)),
    ]
    args = [emb_c, W, b.reshape(1, H), g.reshape(1, H), be.reshape(1, H),
            eids_c, etab]
    aliases = {}
    if aliased:
        in_specs = [pl.BlockSpec(memory_space=pl.ANY),
                    pl.BlockSpec(memory_space=pl.ANY)] + in_specs
        args = [node_acc, eattr_acc] + args
        aliases = {0: 0, 1: 1}
    return pl.pallas_call(
        body,
        grid=(nsteps,),
        in_specs=in_specs,
        out_specs=[
            pl.BlockSpec((_BLK, H), lambda i: (base + i, 0)),
            pl.BlockSpec((_BLK, EA), lambda i: (base + i, 0)),
        ],
        out_shape=[
            jax.ShapeDtypeStruct((Btot, H), jnp.float32),
            jax.ShapeDtypeStruct((Btot, EA), jnp.float32),
        ],
        input_output_aliases=aliases,
    )(*args)


def kernel(value_ids, edge_type_ids, value_table, W_proj, b_proj, ln_gamma,
           ln_beta, edge_type_table):
    B = value_ids.shape[0]
    E = edge_type_ids.shape[0]
    H = value_table.shape[1]
    Bc = B // _NCHUNKS
    Ec = E // _NCHUNKS
    vids = value_ids.astype(jnp.int32)
    eids = edge_type_ids.astype(jnp.int32)

    embs = []
    for c in range(_NCHUNKS):
        vids2 = lax.slice(vids, (c * Bc,), ((c + 1) * Bc,)).reshape(
            Bc // _VCHUNK, _VCHUNK)
        embs.append(_sc_gather(value_table, vids2, Bc, H))

    node_acc, eattr_acc = None, None
    for c in range(_NCHUNKS):
        eids_c = lax.slice(eids, (c * Ec,), ((c + 1) * Ec,)).reshape(
            Ec // _BLK, 1, _BLK)
        node_acc, eattr_acc = _tc_dense(
            embs[c], W_proj, b_proj, ln_gamma, ln_beta, eids_c,
            edge_type_table, node_acc, eattr_acc, c, B)
    return node_acc, eattr_acc
